# baseline (speedup 1.0000x reference)
import numpy as np
import jax
from jax import lax
import jax.numpy as jnp
from jax.experimental import pallas as pl
from jax.experimental.pallas import tpu as pltpu
from jax.experimental.pallas import tpu_sc as plsc

_R = 32
_V = 1_000_000
_SUPPRESS = 4
_BLK = 64
_NBLK = _V // _BLK
_CHUNK = 65536
_K1_STEPS = 16
_NBLK_PAD = _K1_STEPS * (_CHUNK // _BLK)
_GRP = 128
_NGRP = _NBLK_PAD // _GRP
_K = 50
_IDX_PAD = 64
_VBLK = 128
_NC = 2
_NS = 16

_KEY0 = np.uint32(0)
_KEY1 = np.uint32(42)
_NEG_INF = np.float32(-np.inf)


def _threefry_bits(x1):
    ks0, ks1 = _KEY0, _KEY1
    ks2 = np.uint32(ks0 ^ ks1 ^ np.uint32(0x1BD11BDA))
    ks = (ks0, ks1, ks2)
    rots = ((13, 15, 26, 6), (17, 29, 16, 24))
    x0 = jnp.full_like(x1, ks0)
    x1 = x1 + ks1
    for i in range(5):
        for r in rots[i % 2]:
            x0 = x0 + x1
            x1 = (x1 << np.uint32(r)) | (x1 >> np.uint32(32 - r))
            x1 = x1 ^ x0
        x0 = x0 + ks[(i + 1) % 3]
        x1 = x1 + np.uint32(ks[(i + 2) % 3] + np.uint32(i + 1))
    return x0 ^ x1


def _gumbel(lin_idx_u32):
    bits = _threefry_bits(lin_idx_u32)
    fb = (bits >> np.uint32(9)) | np.uint32(0x3F800000)
    f = jax.lax.bitcast_convert_type(fb, jnp.float32) - jnp.float32(1.0)
    tiny = jnp.float32(np.finfo(np.float32).tiny)
    u = jnp.maximum(tiny, f * (jnp.float32(1.0) - tiny) + tiny)
    return -jnp.log(-jnp.log(u))


def _k1_blockmax(x_ref, o_ref):
    i = pl.program_id(0)
    edge = (i == 0) | (i == _K1_STEPS - 1)

    @pl.when(edge)
    def _():
        col = jax.lax.broadcasted_iota(jnp.int32, (_R, _CHUNK), 1) + i * _CHUNK
        x = jnp.where((col < _V) & (col >= _SUPPRESS), x_ref[...], _NEG_INF)
        o_ref[...] = jnp.max(x.reshape(_R, _CHUNK // _BLK, _BLK), axis=2)

    @pl.when(jnp.logical_not(edge))
    def _():
        o_ref[...] = jnp.max(
            x_ref[...].reshape(_R, _CHUNK // _BLK, _BLK), axis=2)


def _k2a_topgroups(bm_ref, gid_ref):
    l2 = jnp.max(bm_ref[...].reshape(_R, _NGRP, _GRP), axis=2)
    lane = jax.lax.broadcasted_iota(jnp.int32, (_R, _NGRP), 1)
    lane_o = jax.lax.broadcasted_iota(jnp.int32, (_R, _IDX_PAD), 1)
    rowbase = jax.lax.broadcasted_iota(jnp.int32, (_R, _IDX_PAD), 0) * _NGRP

    def body(j, carry):
        x, acc = carry
        m = jnp.max(x, axis=1, keepdims=True)
        pos = jnp.min(jnp.where(x == m, lane, _NGRP), axis=1, keepdims=True)
        x = jnp.where(lane == pos, _NEG_INF, x)
        return x, jnp.where(lane_o == j, pos + rowbase, acc)

    _, acc = jax.lax.fori_loop(
        0, _K, body, (l2, jnp.zeros((_R, _IDX_PAD), jnp.int32)))
    gid_ref[...] = acc


def _sc_gather_body(idx_hbm, table_hbm, out_hbm, idx_v, rows_v, sem):
    wid = lax.axis_index("s") * _NC + lax.axis_index("c")
    pltpu.sync_copy(idx_hbm.at[wid], idx_v)
    pltpu.async_copy(table_hbm.at[idx_v], rows_v, sem).wait()
    pltpu.sync_copy(rows_v, out_hbm.at[wid])


def _sc_gather(idx, table):
    seg = table.shape[-1]
    mesh = plsc.VectorSubcoreMesh(core_axis_name="c", subcore_axis_name="s")
    return pl.kernel(
        _sc_gather_body,
        out_type=jax.ShapeDtypeStruct((_R, _IDX_PAD, seg), table.dtype),
        mesh=mesh,
        scratch_types=[
            pltpu.VMEM((_IDX_PAD,), jnp.int32),
            pltpu.VMEM((_IDX_PAD, seg), table.dtype),
            pltpu.SemaphoreType.DMA,
        ],
    )(idx, table)


def _k2b_topblocks(seg_ref, gidf_ref, idx_ref):
    imax = jnp.int32(2**31 - 1)
    rowi = jax.lax.broadcasted_iota(jnp.int32, (_R, _IDX_PAD), 0)
    gid = gidf_ref[...] - rowi * _NGRP
    jj = jax.lax.broadcasted_iota(jnp.int32, (_R, _IDX_PAD, _GRP), 1)
    lane2 = jax.lax.broadcasted_iota(jnp.int32, (_R, _IDX_PAD, _GRP), 2)
    blockid = (gid[:, :, None] * _GRP + lane2).reshape(_R, _IDX_PAD * _GRP)
    x = jnp.where(jj < _K, seg_ref[...], _NEG_INF).reshape(
        _R, _IDX_PAD * _GRP)
    flat = jax.lax.broadcasted_iota(jnp.int32, (_R, _IDX_PAD * _GRP), 1)
    lane_o = jax.lax.broadcasted_iota(jnp.int32, (_R, _IDX_PAD), 1)
    rowbase = rowi * _NBLK

    def body(j, carry):
        x, acc = carry
        m = jnp.max(x, axis=1, keepdims=True)
        pos = jnp.min(jnp.where(x == m, flat, imax), axis=1, keepdims=True)
        bid = jnp.min(jnp.where(flat == pos, blockid, imax), axis=1,
                      keepdims=True)
        x = jnp.where(flat == pos, _NEG_INF, x)
        return x, jnp.where(lane_o == j, (bid + rowbase) >> 1, acc)

    _, acc = jax.lax.fori_loop(
        0, _K, body, (x, jnp.zeros((_R, _IDX_PAD), jnp.int32)))
    idx_ref[...] = acc


def _k3_sample(val_ref, fbid_ref, temp_ref, tk_ref, tok_ref):
    imax = jnp.int32(2**31 - 1)
    fbid = fbid_ref[...]
    rowi = jax.lax.broadcasted_iota(jnp.int32, (_R, _IDX_PAD), 0)
    jj = jax.lax.broadcasted_iota(jnp.int32, (_R, _IDX_PAD, _VBLK), 1)
    lane2 = jax.lax.broadcasted_iota(jnp.int32, (_R, _IDX_PAD, _VBLK), 2)
    gidx = (fbid[:, :, None] * _VBLK + lane2).reshape(_R, _IDX_PAD * _VBLK)
    col = gidx - jax.lax.broadcasted_iota(
        jnp.int32, (_R, _IDX_PAD * _VBLK), 0) * _V
    fa = jnp.broadcast_to(fbid[:, :, None], (_R, _IDX_PAD, _IDX_PAD))
    fb = jnp.broadcast_to(fbid[:, None, :], (_R, _IDX_PAD, _IDX_PAD))
    tri = (jax.lax.broadcasted_iota(jnp.int32, (_R, _IDX_PAD, _IDX_PAD), 2)
           < jax.lax.broadcasted_iota(jnp.int32, (_R, _IDX_PAD, _IDX_PAD), 1))
    dupi = jnp.sum(jnp.where((fa == fb) & tri, 1, 0), axis=2)
    dup3 = jnp.broadcast_to(dupi[:, :, None], (_R, _IDX_PAD, _VBLK))
    keep3 = (jj < _K) & (dup3 == 0)
    v3 = jnp.where(keep3, val_ref[...], _NEG_INF)
    temp = temp_ref[...]
    v = jnp.where((col < _V) & (col >= _SUPPRESS),
                  v3.reshape(_R, _IDX_PAD * _VBLK) / temp, _NEG_INF)

    flat = jax.lax.broadcasted_iota(jnp.int32, (_R, _IDX_PAD * _VBLK), 1)
    lane_o = jax.lax.broadcasted_iota(jnp.int32, (_R, _IDX_PAD), 1)

    def body(t, carry):
        x, accm = carry
        m = jnp.max(x, axis=1, keepdims=True)
        pos = jnp.min(jnp.where(x == m, flat, imax), axis=1, keepdims=True)
        x = jnp.where(flat == pos, _NEG_INF, x)
        return x, jnp.where(lane_o == t, m, accm)

    _, accm = jax.lax.fori_loop(
        0, _K, body, (v, jnp.full((_R, _IDX_PAD), _NEG_INF)))
    tkm1 = tk_ref[0] - 1
    thresh = jnp.max(jnp.where(lane_o == tkm1, accm, _NEG_INF), axis=1,
                     keepdims=True)

    g = _gumbel(gidx.astype(jnp.uint32))
    w = jnp.where(v >= thresh, v + g, _NEG_INF)
    wm = jnp.max(w, axis=1, keepdims=True)
    win = jnp.min(jnp.where(w == wm, col, imax), axis=1, keepdims=True)
    tok_ref[...] = jnp.broadcast_to(win, (_R, _IDX_PAD))


def kernel(logits, temperatures, top_k):
    l1max = pl.pallas_call(
        _k1_blockmax,
        grid=(_K1_STEPS,),
        in_specs=[pl.BlockSpec((_R, _CHUNK), lambda i: (0, i))],
        out_specs=pl.BlockSpec((_R, _CHUNK // _BLK), lambda i: (0, i)),
        out_shape=jax.ShapeDtypeStruct((_R, _NBLK_PAD), jnp.float32),
    )(logits)

    gid = pl.pallas_call(
        _k2a_topgroups,
        in_specs=[pl.BlockSpec((_R, _NBLK_PAD), lambda: (0, 0))],
        out_specs=pl.BlockSpec((_R, _IDX_PAD), lambda: (0, 0)),
        out_shape=jax.ShapeDtypeStruct((_R, _IDX_PAD), jnp.int32),
    )(l1max)

    segs = _sc_gather(gid, l1max.reshape(_R * _NGRP, _GRP))

    blkidx = pl.pallas_call(
        _k2b_topblocks,
        in_specs=[
            pl.BlockSpec((_R, _IDX_PAD, _GRP), lambda: (0, 0, 0)),
            pl.BlockSpec((_R, _IDX_PAD), lambda: (0, 0)),
        ],
        out_specs=pl.BlockSpec((_R, _IDX_PAD), lambda: (0, 0)),
        out_shape=jax.ShapeDtypeStruct((_R, _IDX_PAD), jnp.int32),
    )(segs, gid)

    vals = _sc_gather(blkidx, logits.reshape(_R * _V // _VBLK, _VBLK))

    tk = jnp.asarray(top_k, jnp.int32).reshape(1)
    toks = pl.pallas_call(
        _k3_sample,
        in_specs=[
            pl.BlockSpec((_R, _IDX_PAD, _VBLK), lambda: (0, 0, 0)),
            pl.BlockSpec((_R, _IDX_PAD), lambda: (0, 0)),
            pl.BlockSpec((_R, 1), lambda: (0, 0)),
            pl.BlockSpec(memory_space=pltpu.SMEM),
        ],
        out_specs=pl.BlockSpec((_R, _IDX_PAD), lambda: (0, 0)),
        out_shape=jax.ShapeDtypeStruct((_R, _IDX_PAD), jnp.int32),
    )(vals, blkidx, temperatures.reshape(_R, 1), tk)

    return toks[:, 0]


# speedup vs baseline: 3.4205x; 3.4205x over previous
import numpy as np
import jax
import jax.numpy as jnp
from jax.experimental import pallas as pl
from jax.experimental.pallas import tpu as pltpu

_R = 32
_V = 1_000_000
_SUPPRESS = 4
_BLK = 64
_CHUNK = 65536
_K1_STEPS = 16
_NBLK_PAD = _K1_STEPS * (_CHUNK // _BLK)
_K = 50
_IDX_PAD = 64

_KEY0 = np.uint32(0)
_KEY1 = np.uint32(42)
_NEG_INF = np.float32(-np.inf)


def _threefry_bits(x1):
    ks0, ks1 = _KEY0, _KEY1
    ks2 = np.uint32(ks0 ^ ks1 ^ np.uint32(0x1BD11BDA))
    ks = (ks0, ks1, ks2)
    rots = ((13, 15, 26, 6), (17, 29, 16, 24))
    x0 = jnp.full_like(x1, ks0)
    x1 = x1 + ks1
    for i in range(5):
        for r in rots[i % 2]:
            x0 = x0 + x1
            x1 = (x1 << np.uint32(r)) | (x1 >> np.uint32(32 - r))
            x1 = x1 ^ x0
        x0 = x0 + ks[(i + 1) % 3]
        x1 = x1 + np.uint32(ks[(i + 2) % 3] + np.uint32(i + 1))
    return x0 ^ x1


def _gumbel(lin_idx_u32):
    bits = _threefry_bits(lin_idx_u32)
    fb = (bits >> np.uint32(9)) | np.uint32(0x3F800000)
    f = jax.lax.bitcast_convert_type(fb, jnp.float32) - jnp.float32(1.0)
    tiny = jnp.float32(np.finfo(np.float32).tiny)
    u = jnp.maximum(tiny, f * (jnp.float32(1.0) - tiny) + tiny)
    return -jnp.log(-jnp.log(u))


def _k1_blockmax(x_ref, o_ref):
    i = pl.program_id(0)
    edge = (i == 0) | (i == _K1_STEPS - 1)

    @pl.when(edge)
    def _():
        col = jax.lax.broadcasted_iota(jnp.int32, (_R, _CHUNK), 1) + i * _CHUNK
        x = jnp.where((col < _V) & (col >= _SUPPRESS), x_ref[...], _NEG_INF)
        o_ref[...] = jnp.max(x.reshape(_R, _CHUNK // _BLK, _BLK), axis=2)

    @pl.when(jnp.logical_not(edge))
    def _():
        o_ref[...] = jnp.max(
            x_ref[...].reshape(_R, _CHUNK // _BLK, _BLK), axis=2)


def _k2_topblocks(bm_ref, idx_ref, scr_ref):
    scr_ref[...] = bm_ref[...]
    lane = jax.lax.broadcasted_iota(jnp.int32, (_R, _NBLK_PAD), 1)
    lane_o = jax.lax.broadcasted_iota(jnp.int32, (_R, _IDX_PAD), 1)

    def body(j, acc):
        x = scr_ref[...]
        m = jnp.max(x, axis=1, keepdims=True)
        pos = jnp.min(jnp.where(x == m, lane, _NBLK_PAD), axis=1, keepdims=True)
        scr_ref[...] = jnp.where(lane == pos, _NEG_INF, x)
        return jnp.where(lane_o == j, pos, acc)

    idx_ref[...] = jax.lax.fori_loop(
        0, _K, body, jnp.zeros((_R, _IDX_PAD), jnp.int32))


def _k3_sample(idx_s, *refs):
    (blk_refs, bvec_ref, temp_ref, tk_ref, tok_ref, mval_ref) = (
        refs[:_K], refs[_K], refs[_K + 1], refs[_K + 2], refs[_K + 3],
        refs[_K + 4])
    r = pl.program_id(0)
    buf = jnp.concatenate([b[0] for b in blk_refs], axis=0)

    temp = temp_ref[r]
    bvec = bvec_ref[0, 0, :_K].astype(jnp.int32)
    col = bvec[:, None] * _BLK + jax.lax.broadcasted_iota(
        jnp.int32, (_K, _BLK), 1)
    valid = (col < _V) & (col >= _SUPPRESS)
    v = jnp.where(valid, buf / temp, _NEG_INF)

    flat = jax.lax.broadcasted_iota(jnp.int32, (_K, _BLK), 0) * _BLK + \
        jax.lax.broadcasted_iota(jnp.int32, (_K, _BLK), 1)

    def body(t, x):
        m = jnp.max(x)
        mval_ref[t] = m
        pos = jnp.min(jnp.where(x == m, flat, _K * _BLK))
        return jnp.where(flat == pos, _NEG_INF, x)

    jax.lax.fori_loop(0, _K, body, v)
    thresh = mval_ref[tk_ref[0] - 1]

    g = _gumbel((jnp.int32(r * _V) + col).astype(jnp.uint32))
    w = jnp.where(v >= thresh, v + g, _NEG_INF)
    wm = jnp.max(w)
    win = jnp.min(jnp.where(w == wm, col, jnp.int32(2**31 - 1)))
    tok_ref[...] = jnp.full((1, 1, 1), win, jnp.int32)


def kernel(logits, temperatures, top_k):
    l1max = pl.pallas_call(
        _k1_blockmax,
        grid=(_K1_STEPS,),
        in_specs=[pl.BlockSpec((_R, _CHUNK), lambda i: (0, i))],
        out_specs=pl.BlockSpec((_R, _CHUNK // _BLK), lambda i: (0, i)),
        out_shape=jax.ShapeDtypeStruct((_R, _NBLK_PAD), jnp.float32),
    )(logits)

    blkidx = pl.pallas_call(
        _k2_topblocks,
        in_specs=[pl.BlockSpec((_R, _NBLK_PAD), lambda: (0, 0))],
        out_specs=pl.BlockSpec((_R, _IDX_PAD), lambda: (0, 0)),
        out_shape=jax.ShapeDtypeStruct((_R, _IDX_PAD), jnp.int32),
        scratch_shapes=[pltpu.VMEM((_R, _NBLK_PAD), jnp.float32)],
    )(l1max)

    tk = jnp.asarray(top_k, jnp.int32).reshape(1)
    toks = pl.pallas_call(
        _k3_sample,
        grid_spec=pltpu.PrefetchScalarGridSpec(
            num_scalar_prefetch=1,
            grid=(_R,),
            in_specs=[
                pl.BlockSpec(
                    (1, 1, _BLK),
                    (lambda r, idx, _j=j: (r * (_V // _BLK) + idx[r, _j], 0, 0)))
                for j in range(_K)
            ] + [
                pl.BlockSpec((1, 1, _IDX_PAD), lambda r, idx: (r, 0, 0)),
                pl.BlockSpec(memory_space=pltpu.SMEM),
                pl.BlockSpec(memory_space=pltpu.SMEM),
            ],
            out_specs=pl.BlockSpec((1, 1, 1), lambda r, idx: (r, 0, 0)),
            scratch_shapes=[
                pltpu.SMEM((_IDX_PAD,), jnp.float32),
            ],
        ),
        out_shape=jax.ShapeDtypeStruct((_R, 1, 1), jnp.int32),
    )(blkidx,
      *([logits.reshape(_R * (_V // _BLK), 1, _BLK)] * _K),
      blkidx.reshape(_R, 1, _IDX_PAD), temperatures, tk)
    return toks[:, 0, 0]
